# Initial kernel scaffold; baseline (speedup 1.0000x reference)
import jax
import jax.numpy as jnp
from jax.experimental import pallas as pl


def kernel(word_ids, edge_src, edge_dst, ws_link, sent_feature, word_table, ws_table, W_sent, p_w2s, p_s2w):
    raise NotImplementedError("write your pallas kernel here")


# speedup vs baseline: 1.1682x; 1.1682x over previous
import functools
import jax
import jax.numpy as jnp
from jax import lax
from jax.experimental import pallas as pl
from jax.experimental.pallas import tpu as pltpu, tpu_sc as plsc

N_WORDS = 10000
N_SENTS = 500
N_EDGES = 320000
VOCAB = 50000
EMBED = 128
HID = 64
EDGE_EMB = 50
LSTM2 = 512
FFN = 256
N_ITER = 2
N_ETYPES = 40

_NC, _NS, _L = 2, 16, 16
_NW = _NC * _NS


def _sc_gather_rows(table, idx, n_rows, d):
    b_pad = ((n_rows + 8 * _NW - 1) // (8 * _NW)) * (8 * _NW)
    idx_p = jnp.zeros((b_pad,), jnp.int32).at[:n_rows].set(idx)
    b_per_w = b_pad // _NW
    mesh = plsc.VectorSubcoreMesh(core_axis_name="c", subcore_axis_name="s")

    @functools.partial(
        pl.kernel, mesh=mesh,
        out_type=jax.ShapeDtypeStruct((b_pad, d), jnp.float32),
        scratch_types=[
            pltpu.VMEM((b_per_w,), jnp.int32),
            pltpu.VMEM((b_per_w, d), jnp.float32),
            pltpu.SemaphoreType.DMA,
        ],
    )
    def k(table_hbm, idx_hbm, out_hbm, idx_v, rows_v, sem):
        wid = lax.axis_index("s") * _NC + lax.axis_index("c")
        base = wid * b_per_w
        pltpu.sync_copy(idx_hbm.at[pl.ds(base, b_per_w)], idx_v)
        pltpu.async_copy(table_hbm.at[idx_v], rows_v, sem).wait()
        pltpu.sync_copy(rows_v, out_hbm.at[pl.ds(base, b_per_w)])

    return k(table, idx_p)[:n_rows]


def _gat_pass(src_state, dst_state, p, src_idx, dst_idx, ws_link, ws_table, n_dst):
    zs = src_state @ p["Wsrc"]
    ze_tab = ws_table @ p["We"]
    ls = zs @ p["a_src"]
    ld = dst_state @ (p["Wdst"] @ p["a_dst"])
    le = ze_tab @ p["a_e"]
    logits = jax.nn.leaky_relu(ls[src_idx] + ld[dst_idx] + le[ws_link], 0.2)
    ex = jnp.exp(logits)
    den = jax.ops.segment_sum(ex, dst_idx, num_segments=n_dst)
    agg = jax.ops.segment_sum(
        ex[:, None] * (zs[src_idx] + ze_tab[ws_link]), dst_idx, num_segments=n_dst)
    h = jax.nn.elu(agg / (den + 1e-9)[:, None])
    ffn = jax.nn.relu(h @ p["Wf1"]) @ p["Wf2"]
    return h + ffn


def kernel(word_ids, edge_src, edge_dst, ws_link, sent_feature, word_table, ws_table, W_sent, p_w2s, p_s2w):
    word_state = _sc_gather_rows(word_table, word_ids, N_WORDS, EMBED)
    sent_state = sent_feature @ W_sent
    word_state = _gat_pass(sent_state, word_state, p_s2w, edge_dst, edge_src,
                           ws_link, ws_table, N_WORDS)
    for _ in range(N_ITER):
        sent_state = _gat_pass(word_state, sent_state, p_w2s, edge_src, edge_dst,
                               ws_link, ws_table, N_SENTS)
        word_state = _gat_pass(sent_state, word_state, p_s2w, edge_dst, edge_src,
                               ws_link, ws_table, N_WORDS)
    return word_state


# speedup vs baseline: 5.0420x; 4.3160x over previous
import functools
import jax
import jax.numpy as jnp
from jax import lax
from jax.experimental import pallas as pl
from jax.experimental.pallas import tpu as pltpu, tpu_sc as plsc

N_WORDS = 10000
N_SENTS = 500
N_EDGES = 320000
VOCAB = 50000
EMBED = 128
HID = 64
LSTM2 = 512
FFN_D = 256
N_ITER = 2
N_ETYPES = 40

_NC, _NS = 2, 16
_NW = _NC * _NS
_C = 80
_EPW = N_EDGES // _NW
_NCHUNK = _EPW // _C


def _sc_gather_rows(table, idx, n_rows, d):
    b_pad = ((n_rows + 8 * _NW - 1) // (8 * _NW)) * (8 * _NW)
    idx_p = jnp.zeros((b_pad,), jnp.int32).at[:n_rows].set(idx)
    b_per_w = b_pad // _NW
    mesh = plsc.VectorSubcoreMesh(core_axis_name="c", subcore_axis_name="s")

    @functools.partial(
        pl.kernel, mesh=mesh,
        out_type=jax.ShapeDtypeStruct((b_pad, d), jnp.float32),
        scratch_types=[
            pltpu.VMEM((b_per_w,), jnp.int32),
            pltpu.VMEM((b_per_w, d), jnp.float32),
            pltpu.SemaphoreType.DMA,
        ],
    )
    def k(table_hbm, idx_hbm, out_hbm, idx_v, rows_v, sem):
        wid = lax.axis_index("s") * _NC + lax.axis_index("c")
        base = wid * b_per_w
        pltpu.sync_copy(idx_hbm.at[pl.ds(base, b_per_w)], idx_v)
        pltpu.async_copy(table_hbm.at[idx_v], rows_v, sem).wait()
        pltpu.sync_copy(rows_v, out_hbm.at[pl.ds(base, b_per_w)])

    return k(table, idx_p)[:n_rows]


_GW = 128


def _make_edge_sc(n_src, ls_pad, ld_pad, agg_pad, out, nsplit):
    rows = agg_pad // nsplit
    stripe = rows // _NS
    n_work = _NW // nsplit
    epw = N_EDGES // n_work
    nchunk = epw // _C
    mesh = plsc.VectorSubcoreMesh(core_axis_name="c", subcore_axis_name="s")
    ncol = out // 16

    @functools.partial(
        pl.kernel, mesh=mesh,
        compiler_params=pltpu.CompilerParams(needs_layout_passes=False),
        out_type=[
            jax.ShapeDtypeStruct((_NC, rows, _GW), jnp.float32),
            jax.ShapeDtypeStruct((_NC, rows, 16), jnp.float32),
        ],
        scratch_types=[
            pltpu.VMEM((ls_pad,), jnp.float32),
            pltpu.VMEM((ld_pad,), jnp.float32),
            pltpu.VMEM((48,), jnp.float32),
            pltpu.VMEM((3 * _C,), jnp.int32),
            pltpu.VMEM((_C,), jnp.float32),
            pltpu.VMEM((_C,), jnp.int32),
            pltpu.VMEM((_C,), jnp.int32),
            pltpu.VMEM((_C,), jnp.int32),
            pltpu.VMEM((_C, _GW), jnp.float32),
            pltpu.VMEM((_C, _GW), jnp.float32),
            pltpu.VMEM((_C, 16), jnp.float32),
            pltpu.VMEM_SHARED((rows, _GW), jnp.float32),
            pltpu.VMEM_SHARED((rows, 16), jnp.float32),
            pltpu.SemaphoreType.DMA,
            pltpu.SemaphoreType.DMA,
        ],
    )
    def k(eidx_hbm, u_hbm, ls_hbm, ld_hbm, le_hbm, z1_hbm, z2_hbm,
          out_hbm, den_hbm,
          lstab, ldtab, letab, echunk, exbuf, sbuf, i2buf, dbuf,
          rows_s, rows_e, den_buf, agg_sh, den_sh, sem1, sem2):
        cid = lax.axis_index("c")
        sid = lax.axis_index("s")
        wid = sid * _NC + cid if nsplit == 1 else sid
        lo = jnp.int32(0) if nsplit == 1 else cid * jnp.int32(rows)
        pltpu.sync_copy(ls_hbm, lstab)
        pltpu.sync_copy(ld_hbm, ldtab)
        pltpu.sync_copy(le_hbm, letab)
        pltpu.sync_copy(z1_hbm.at[pl.ds(sid * stripe, stripe)],
                        agg_sh.at[pl.ds(sid * stripe, stripe)])
        pltpu.sync_copy(z2_hbm.at[pl.ds(sid * stripe, stripe)],
                        den_sh.at[pl.ds(sid * stripe, stripe)])
        plsc.subcore_barrier()

        ebase = wid * (nchunk * 3 * _C)

        def p2(c, _):
            pltpu.sync_copy(
                eidx_hbm.at[pl.ds(ebase + c * (3 * _C), 3 * _C)], echunk)

            def g1(g, _):
                sl = pl.ds(g * 16, 16)
                sv = echunk[pl.ds(g * 16, 16)]
                dv = echunk[pl.ds(_C + g * 16, 16)]
                tv = echunk[pl.ds(2 * _C + g * 16, 16)]
                av = plsc.load_gather(lstab, [sv])
                bv = plsc.load_gather(ldtab, [dv])
                cv = plsc.load_gather(letab, [tv])
                l = av + bv + cv
                l = jnp.where(l > 0, l, l * jnp.float32(0.2))
                e = jnp.exp(l)
                dl = dv - lo
                if nsplit > 1:
                    inr = (dl >= 0) & (dl < rows)
                    e = jnp.where(inr, e, jnp.float32(0.0))
                    dl = jnp.clip(dl, 0, rows - 1)
                exbuf[sl] = e
                sbuf[sl] = sv
                dbuf[sl] = dl
                i2buf[sl] = tv + jnp.int32(n_src)
                return 0
            lax.fori_loop(0, _C // 16, g1, 0)
            pltpu.async_copy(u_hbm.at[sbuf], rows_s, sem1).wait()
            pltpu.async_copy(u_hbm.at[i2buf], rows_e, sem2).wait()

            def wrow(i, _):
                exb = plsc.load_gather(exbuf, [jnp.full((16,), i, jnp.int32)])
                den_buf[i, :] = exb
                for j in range(ncol):
                    sl = pl.ds(j * 16, 16)
                    rows_s[i, sl] = exb * (rows_s[i, sl] + rows_e[i, sl])
                return 0
            lax.fori_loop(0, _C, wrow, 0)
            pltpu.sync_copy(rows_s, agg_sh.at[dbuf], add=True)
            pltpu.sync_copy(den_buf, den_sh.at[dbuf], add=True)
            return 0
        lax.fori_loop(0, nchunk, p2, 0)

        plsc.subcore_barrier()
        pltpu.sync_copy(agg_sh.at[pl.ds(sid * stripe, stripe)],
                        out_hbm.at[cid, pl.ds(sid * stripe, stripe)])
        pltpu.sync_copy(den_sh.at[pl.ds(sid * stripe, stripe)],
                        den_hbm.at[cid, pl.ds(sid * stripe, stripe)])

    return k


def _make_prep_tc(n_src, n_dst, ls_pad, ld_pad, rows_pad, out, out_pad):
    del out_pad
    def body(src_ref, dst_ref, wsrc_ref, wdst_ref, we_ref, asrc_ref,
             adst_ref, ae_ref, wst_ref, u_ref, ls_ref, ld_ref, le_ref):
        zs = jnp.dot(src_ref[...], wsrc_ref[...],
                     preferred_element_type=jnp.float32)
        zet = jnp.dot(wst_ref[...], we_ref[...],
                      preferred_element_type=jnp.float32)
        ls = jnp.dot(zs, asrc_ref[...][:, None],
                     preferred_element_type=jnp.float32)
        wv = jnp.dot(wdst_ref[...], adst_ref[...][:, None],
                     preferred_element_type=jnp.float32)
        ld = jnp.dot(dst_ref[...], wv, preferred_element_type=jnp.float32)
        le = jnp.dot(zet, ae_ref[...][:, None],
                     preferred_element_type=jnp.float32)
        if out < _GW:
            zs = jnp.concatenate(
                [zs, jnp.zeros((n_src, _GW - out), jnp.float32)], axis=1)
            zet = jnp.concatenate(
                [zet, jnp.zeros((N_ETYPES, _GW - out), jnp.float32)], axis=1)
        parts = [zs, zet]
        extra = rows_pad - n_src - N_ETYPES
        if extra:
            parts.append(jnp.zeros((extra, _GW), jnp.float32))
        u_ref[...] = jnp.concatenate(parts, axis=0)

        def padrows(x, total):
            extra = total - x.shape[0]
            if not extra:
                return x
            return jnp.concatenate(
                [x, jnp.zeros((extra, 1), jnp.float32)], axis=0)

        ls_ref[...] = padrows(ls, ls_pad)
        ld_ref[...] = padrows(ld, ld_pad)
        le_ref[...] = padrows(le, 48)

    return pl.pallas_call(
        body,
        out_shape=[
            jax.ShapeDtypeStruct((rows_pad, _GW), jnp.float32),
            jax.ShapeDtypeStruct((ls_pad, 1), jnp.float32),
            jax.ShapeDtypeStruct((ld_pad, 1), jnp.float32),
            jax.ShapeDtypeStruct((48, 1), jnp.float32),
        ],
    )


def _make_finish_tc(n_dst, agg_pad, out, nsplit):
    def body(agg_ref, den_ref, wf1_ref, wf2_ref, out_ref):
        if nsplit == 1:
            a = agg_ref[0] + agg_ref[1]
            den = (den_ref[0] + den_ref[1])[:, 0:1]
        else:
            a = jnp.concatenate([agg_ref[0], agg_ref[1]], axis=0)
            den = jnp.concatenate([den_ref[0], den_ref[1]], axis=0)[:, 0:1]
        g = a[:, :out] / (den + jnp.float32(1e-9))
        h = jnp.where(g > 0, g, jnp.exp(g) - jnp.float32(1.0))
        f = jnp.maximum(
            jnp.dot(h, wf1_ref[...], preferred_element_type=jnp.float32),
            jnp.float32(0.0))
        out_ref[...] = h + jnp.dot(f, wf2_ref[...],
                                   preferred_element_type=jnp.float32)

    return pl.pallas_call(
        body,
        out_shape=jax.ShapeDtypeStruct((agg_pad, out), jnp.float32))


def _sent_proj(sent_feature, W_sent):
    def body(x_ref, w_ref, o_ref):
        o_ref[...] = jnp.dot(x_ref[...], w_ref[...],
                             preferred_element_type=jnp.float32)
    return pl.pallas_call(
        body, out_shape=jax.ShapeDtypeStruct((N_SENTS, HID), jnp.float32)
    )(sent_feature, W_sent)


_S2W = dict(n_src=N_SENTS, n_dst=N_WORDS, ls_pad=512, ld_pad=N_WORDS,
            agg_pad=10240, rows_pad=544, out=EMBED, nsplit=2)
_W2S = dict(n_src=N_WORDS, n_dst=N_SENTS, ls_pad=N_WORDS, ld_pad=512,
            agg_pad=512, rows_pad=N_WORDS + 48, out=HID, nsplit=1)

_EDGE_SC = {k: _make_edge_sc(c["n_src"], c["ls_pad"], c["ld_pad"],
                             c["agg_pad"], c["out"], c["nsplit"])
            for k, c in (("s2w", _S2W), ("w2s", _W2S))}
_PREP_TC = {k: _make_prep_tc(c["n_src"], c["n_dst"], c["ls_pad"], c["ld_pad"],
                             c["rows_pad"], c["out"], None)
            for k, c in (("s2w", _S2W), ("w2s", _W2S))}
_FIN_TC = {k: _make_finish_tc(c["n_dst"], c["agg_pad"], c["out"], c["nsplit"])
           for k, c in (("s2w", _S2W), ("w2s", _W2S))}


def _edge_slab(src_idx, dst_idx, lnk, nsplit):
    n_work = _NW // nsplit
    nchunk = N_EDGES // n_work // _C
    e = jnp.stack([src_idx, dst_idx, lnk], axis=0)
    e = e.reshape(3, n_work, nchunk, _C).transpose(1, 2, 0, 3)
    return e.reshape(-1)


def _gat_pass(kind, cfg, src_state, dst_state, p, eidx, z1, z2, ws_table):
    u, ls, ld, le = _PREP_TC[kind](
        src_state, dst_state, p["Wsrc"], p["Wdst"], p["We"],
        p["a_src"], p["a_dst"], p["a_e"], ws_table)
    agg2, den2 = _EDGE_SC[kind](eidx, u, ls.reshape(-1), ld.reshape(-1),
                                le.reshape(-1), z1, z2)
    out = _FIN_TC[kind](agg2, den2, p["Wf1"], p["Wf2"])
    return out[:cfg["n_dst"]]


def kernel(word_ids, edge_src, edge_dst, ws_link, sent_feature, word_table,
           ws_table, W_sent, p_w2s, p_s2w):
    word_state = _sc_gather_rows(word_table, word_ids, N_WORDS, EMBED)
    sent_state = _sent_proj(sent_feature, W_sent)
    eidx_w2s = _edge_slab(edge_src, edge_dst, ws_link, _W2S["nsplit"])
    eidx_s2w = _edge_slab(edge_dst, edge_src, ws_link, _S2W["nsplit"])
    z1_s2w = jnp.zeros((_S2W["agg_pad"] // _S2W["nsplit"], _GW), jnp.float32)
    z2_s2w = jnp.zeros((_S2W["agg_pad"] // _S2W["nsplit"], 16), jnp.float32)
    z1_w2s = jnp.zeros((_W2S["agg_pad"], _GW), jnp.float32)
    z2_w2s = jnp.zeros((_W2S["agg_pad"], 16), jnp.float32)

    word_state = _gat_pass("s2w", _S2W, sent_state, word_state, p_s2w,
                           eidx_s2w, z1_s2w, z2_s2w, ws_table)

    def it(_, carry):
        ws, ss = carry
        ss = _gat_pass("w2s", _W2S, ws, ss, p_w2s,
                       eidx_w2s, z1_w2s, z2_w2s, ws_table)
        ws = _gat_pass("s2w", _S2W, ss, ws, p_s2w,
                       eidx_s2w, z1_s2w, z2_s2w, ws_table)
        return (ws, ss)

    word_state, sent_state = lax.fori_loop(
        0, N_ITER, it, (word_state, sent_state))
    return word_state


# speedup vs baseline: 5.6218x; 1.1150x over previous
import functools
import jax
import jax.numpy as jnp
from jax import lax
from jax.experimental import pallas as pl
from jax.experimental.pallas import tpu as pltpu, tpu_sc as plsc

N_WORDS = 10000
N_SENTS = 500
N_EDGES = 320000
VOCAB = 50000
EMBED = 128
HID = 64
LSTM2 = 512
FFN_D = 256
N_ITER = 2
N_ETYPES = 40

_NC, _NS = 2, 16
_NW = _NC * _NS
_C = 80
_EPW = N_EDGES // _NW
_NCHUNK = _EPW // _C


def _sc_gather_rows(table, idx, n_rows, d):
    b_pad = ((n_rows + 8 * _NW - 1) // (8 * _NW)) * (8 * _NW)
    idx_p = jnp.zeros((b_pad,), jnp.int32).at[:n_rows].set(idx)
    b_per_w = b_pad // _NW
    mesh = plsc.VectorSubcoreMesh(core_axis_name="c", subcore_axis_name="s")

    @functools.partial(
        pl.kernel, mesh=mesh,
        out_type=jax.ShapeDtypeStruct((b_pad, d), jnp.float32),
        scratch_types=[
            pltpu.VMEM((b_per_w,), jnp.int32),
            pltpu.VMEM((b_per_w, d), jnp.float32),
            pltpu.SemaphoreType.DMA,
        ],
    )
    def k(table_hbm, idx_hbm, out_hbm, idx_v, rows_v, sem):
        wid = lax.axis_index("s") * _NC + lax.axis_index("c")
        base = wid * b_per_w
        pltpu.sync_copy(idx_hbm.at[pl.ds(base, b_per_w)], idx_v)
        pltpu.async_copy(table_hbm.at[idx_v], rows_v, sem).wait()
        pltpu.sync_copy(rows_v, out_hbm.at[pl.ds(base, b_per_w)])

    return k(table, idx_p)[:n_rows]


_GW = 128


def _make_edge_sc(n_src, ls_pad, ld_pad, agg_pad, out, nsplit):
    rows = agg_pad // nsplit
    stripe = rows // _NS
    n_work = _NW // nsplit
    epw = N_EDGES // n_work
    nchunk = epw // _C
    mesh = plsc.VectorSubcoreMesh(core_axis_name="c", subcore_axis_name="s")
    ncol = out // 16

    @functools.partial(
        pl.kernel, mesh=mesh,
        compiler_params=pltpu.CompilerParams(needs_layout_passes=False),
        out_type=[
            jax.ShapeDtypeStruct((_NC, rows, _GW), jnp.float32),
            jax.ShapeDtypeStruct((_NC, rows, 16), jnp.float32),
        ],
        scratch_types=[
            pltpu.VMEM((ls_pad,), jnp.float32),
            pltpu.VMEM((ld_pad,), jnp.float32),
            pltpu.VMEM((48,), jnp.float32),
            pltpu.VMEM((3 * _C,), jnp.int32),
            pltpu.VMEM((_C,), jnp.float32),
            pltpu.VMEM((_C,), jnp.int32),
            pltpu.VMEM((_C,), jnp.int32),
            pltpu.VMEM((_C,), jnp.int32),
            pltpu.VMEM((_C, _GW), jnp.float32),
            pltpu.VMEM((_C, _GW), jnp.float32),
            pltpu.VMEM((_C, 16), jnp.float32),
            pltpu.VMEM_SHARED((rows, _GW), jnp.float32),
            pltpu.VMEM_SHARED((rows, 16), jnp.float32),
            pltpu.SemaphoreType.DMA,
            pltpu.SemaphoreType.DMA,
        ],
    )
    def k(eidx_hbm, u_hbm, ls_hbm, ld_hbm, le_hbm, z1_hbm, z2_hbm,
          out_hbm, den_hbm,
          lstab, ldtab, letab, echunk, exbuf, sbuf, i2buf, dbuf,
          rows_s, rows_e, den_buf, agg_sh, den_sh, sem1, sem2):
        cid = lax.axis_index("c")
        sid = lax.axis_index("s")
        wid = sid * _NC + cid if nsplit == 1 else sid
        lo = jnp.int32(0) if nsplit == 1 else cid * jnp.int32(rows)
        pltpu.sync_copy(ls_hbm, lstab)
        pltpu.sync_copy(ld_hbm, ldtab)
        pltpu.sync_copy(le_hbm, letab)
        pltpu.sync_copy(z1_hbm.at[pl.ds(sid * stripe, stripe)],
                        agg_sh.at[pl.ds(sid * stripe, stripe)])
        pltpu.sync_copy(z2_hbm.at[pl.ds(sid * stripe, stripe)],
                        den_sh.at[pl.ds(sid * stripe, stripe)])
        plsc.subcore_barrier()

        ebase = wid * (nchunk * 3 * _C)

        def p2(c, _):
            pltpu.sync_copy(
                eidx_hbm.at[pl.ds(ebase + c * (3 * _C), 3 * _C)], echunk)

            def g1(g, _):
                sl = pl.ds(g * 16, 16)
                sv = echunk[pl.ds(g * 16, 16)]
                dv = echunk[pl.ds(_C + g * 16, 16)]
                tv = echunk[pl.ds(2 * _C + g * 16, 16)]
                av = plsc.load_gather(lstab, [sv])
                bv = plsc.load_gather(ldtab, [dv])
                cv = plsc.load_gather(letab, [tv])
                l = av + bv + cv
                l = jnp.where(l > 0, l, l * jnp.float32(0.2))
                e = jnp.exp(l)
                dl = dv - lo
                if nsplit > 1:
                    inr = (dl >= 0) & (dl < rows)
                    e = jnp.where(inr, e, jnp.float32(0.0))
                    dl = jnp.clip(dl, 0, rows - 1)
                exbuf[sl] = e
                sbuf[sl] = sv
                dbuf[sl] = dl
                i2buf[sl] = tv + jnp.int32(n_src)
                return 0
            lax.fori_loop(0, _C // 16, g1, 0)

            d1 = pltpu.async_copy(u_hbm.at[sbuf], rows_s, sem1)
            d2 = pltpu.async_copy(u_hbm.at[i2buf], rows_e, sem2)
            d1.wait()
            d2.wait()

            def wrow(i, _):
                exb = plsc.load_gather(exbuf, [jnp.full((16,), i, jnp.int32)])
                den_buf[i, :] = exb
                for j in range(ncol):
                    sl = pl.ds(j * 16, 16)
                    rows_s[i, sl] = exb * (rows_s[i, sl] + rows_e[i, sl])
                return 0
            lax.fori_loop(0, _C, wrow, 0)
            pltpu.sync_copy(rows_s, agg_sh.at[dbuf], add=True)
            pltpu.sync_copy(den_buf, den_sh.at[dbuf], add=True)
            return 0
        lax.fori_loop(0, nchunk, p2, 0)

        plsc.subcore_barrier()
        pltpu.sync_copy(agg_sh.at[pl.ds(sid * stripe, stripe)],
                        out_hbm.at[cid, pl.ds(sid * stripe, stripe)])
        pltpu.sync_copy(den_sh.at[pl.ds(sid * stripe, stripe)],
                        den_hbm.at[cid, pl.ds(sid * stripe, stripe)])

    return k


def _make_prep_tc(n_src, n_dst, ls_pad, ld_pad, rows_pad, out, out_pad):
    del out_pad
    def body(src_ref, dst_ref, wsrc_ref, wdst_ref, we_ref, asrc_ref,
             adst_ref, ae_ref, wst_ref, u_ref, ls_ref, ld_ref, le_ref):
        zs = jnp.dot(src_ref[...], wsrc_ref[...],
                     preferred_element_type=jnp.float32)
        zet = jnp.dot(wst_ref[...], we_ref[...],
                      preferred_element_type=jnp.float32)
        ls = jnp.dot(zs, asrc_ref[...][:, None],
                     preferred_element_type=jnp.float32)
        wv = jnp.dot(wdst_ref[...], adst_ref[...][:, None],
                     preferred_element_type=jnp.float32)
        ld = jnp.dot(dst_ref[...], wv, preferred_element_type=jnp.float32)
        le = jnp.dot(zet, ae_ref[...][:, None],
                     preferred_element_type=jnp.float32)
        if out < _GW:
            zs = jnp.concatenate(
                [zs, jnp.zeros((n_src, _GW - out), jnp.float32)], axis=1)
            zet = jnp.concatenate(
                [zet, jnp.zeros((N_ETYPES, _GW - out), jnp.float32)], axis=1)
        parts = [zs, zet]
        extra = rows_pad - n_src - N_ETYPES
        if extra:
            parts.append(jnp.zeros((extra, _GW), jnp.float32))
        u_ref[...] = jnp.concatenate(parts, axis=0)

        def padrows(x, total):
            extra = total - x.shape[0]
            if not extra:
                return x
            return jnp.concatenate(
                [x, jnp.zeros((extra, 1), jnp.float32)], axis=0)

        ls_ref[...] = padrows(ls, ls_pad)
        ld_ref[...] = padrows(ld, ld_pad)
        le_ref[...] = padrows(le, 48)

    return pl.pallas_call(
        body,
        out_shape=[
            jax.ShapeDtypeStruct((rows_pad, _GW), jnp.float32),
            jax.ShapeDtypeStruct((ls_pad, 1), jnp.float32),
            jax.ShapeDtypeStruct((ld_pad, 1), jnp.float32),
            jax.ShapeDtypeStruct((48, 1), jnp.float32),
        ],
    )


def _make_finish_tc(n_dst, agg_pad, out, nsplit):
    def body(agg_ref, den_ref, wf1_ref, wf2_ref, out_ref):
        if nsplit == 1:
            a = agg_ref[0] + agg_ref[1]
            den = (den_ref[0] + den_ref[1])[:, 0:1]
        else:
            a = jnp.concatenate([agg_ref[0], agg_ref[1]], axis=0)
            den = jnp.concatenate([den_ref[0], den_ref[1]], axis=0)[:, 0:1]
        g = a[:, :out] / (den + jnp.float32(1e-9))
        h = jnp.where(g > 0, g, jnp.exp(g) - jnp.float32(1.0))
        f = jnp.maximum(
            jnp.dot(h, wf1_ref[...], preferred_element_type=jnp.float32),
            jnp.float32(0.0))
        out_ref[...] = h + jnp.dot(f, wf2_ref[...],
                                   preferred_element_type=jnp.float32)

    return pl.pallas_call(
        body,
        out_shape=jax.ShapeDtypeStruct((agg_pad, out), jnp.float32))


def _sent_proj(sent_feature, W_sent):
    def body(x_ref, w_ref, o_ref):
        o_ref[...] = jnp.dot(x_ref[...], w_ref[...],
                             preferred_element_type=jnp.float32)
    return pl.pallas_call(
        body, out_shape=jax.ShapeDtypeStruct((N_SENTS, HID), jnp.float32)
    )(sent_feature, W_sent)


_S2W = dict(n_src=N_SENTS, n_dst=N_WORDS, ls_pad=512, ld_pad=N_WORDS,
            agg_pad=10240, rows_pad=544, out=EMBED, nsplit=2)
_W2S = dict(n_src=N_WORDS, n_dst=N_SENTS, ls_pad=N_WORDS, ld_pad=512,
            agg_pad=512, rows_pad=N_WORDS + 48, out=HID, nsplit=1)

_EDGE_SC = {k: _make_edge_sc(c["n_src"], c["ls_pad"], c["ld_pad"],
                             c["agg_pad"], c["out"], c["nsplit"])
            for k, c in (("s2w", _S2W), ("w2s", _W2S))}
_PREP_TC = {k: _make_prep_tc(c["n_src"], c["n_dst"], c["ls_pad"], c["ld_pad"],
                             c["rows_pad"], c["out"], None)
            for k, c in (("s2w", _S2W), ("w2s", _W2S))}
_FIN_TC = {k: _make_finish_tc(c["n_dst"], c["agg_pad"], c["out"], c["nsplit"])
           for k, c in (("s2w", _S2W), ("w2s", _W2S))}


def _edge_slab(src_idx, dst_idx, lnk, nsplit):
    n_work = _NW // nsplit
    nchunk = N_EDGES // n_work // _C
    e = jnp.stack([src_idx, dst_idx, lnk], axis=0)
    e = e.reshape(3, n_work, nchunk, _C).transpose(1, 2, 0, 3)
    return e.reshape(-1)


def _gat_pass(kind, cfg, src_state, dst_state, p, eidx, z1, z2, ws_table):
    u, ls, ld, le = _PREP_TC[kind](
        src_state, dst_state, p["Wsrc"], p["Wdst"], p["We"],
        p["a_src"], p["a_dst"], p["a_e"], ws_table)
    agg2, den2 = _EDGE_SC[kind](eidx, u, ls.reshape(-1), ld.reshape(-1),
                                le.reshape(-1), z1, z2)
    out = _FIN_TC[kind](agg2, den2, p["Wf1"], p["Wf2"])
    return out[:cfg["n_dst"]]


def kernel(word_ids, edge_src, edge_dst, ws_link, sent_feature, word_table,
           ws_table, W_sent, p_w2s, p_s2w):
    word_state = _sc_gather_rows(word_table, word_ids, N_WORDS, EMBED)
    sent_state = _sent_proj(sent_feature, W_sent)
    eidx_w2s = _edge_slab(edge_src, edge_dst, ws_link, _W2S["nsplit"])
    eidx_s2w = _edge_slab(edge_dst, edge_src, ws_link, _S2W["nsplit"])
    z1_s2w = jnp.zeros((_S2W["agg_pad"] // _S2W["nsplit"], _GW), jnp.float32)
    z2_s2w = jnp.zeros((_S2W["agg_pad"] // _S2W["nsplit"], 16), jnp.float32)
    z1_w2s = jnp.zeros((_W2S["agg_pad"], _GW), jnp.float32)
    z2_w2s = jnp.zeros((_W2S["agg_pad"], 16), jnp.float32)

    word_state = _gat_pass("s2w", _S2W, sent_state, word_state, p_s2w,
                           eidx_s2w, z1_s2w, z2_s2w, ws_table)

    def it(_, carry):
        ws, ss = carry
        ss = _gat_pass("w2s", _W2S, ws, ss, p_w2s,
                       eidx_w2s, z1_w2s, z2_w2s, ws_table)
        ws = _gat_pass("s2w", _S2W, ss, ws, p_s2w,
                       eidx_s2w, z1_s2w, z2_s2w, ws_table)
        return (ws, ss)

    word_state, sent_state = lax.fori_loop(
        0, N_ITER, it, (word_state, sent_state))
    return word_state


# speedup vs baseline: 5.6409x; 1.0034x over previous
import functools
import jax
import jax.numpy as jnp
from jax import lax
from jax.experimental import pallas as pl
from jax.experimental.pallas import tpu as pltpu, tpu_sc as plsc

N_WORDS = 10000
N_SENTS = 500
N_EDGES = 320000
VOCAB = 50000
EMBED = 128
HID = 64
LSTM2 = 512
FFN_D = 256
N_ITER = 2
N_ETYPES = 40

_NC, _NS = 2, 16
_NW = _NC * _NS
_C = 80
_EPW = N_EDGES // _NW
_NCHUNK = _EPW // _C


def _sc_gather_rows(table, idx, n_rows, d):
    b_pad = ((n_rows + 8 * _NW - 1) // (8 * _NW)) * (8 * _NW)
    idx_p = jnp.zeros((b_pad,), jnp.int32).at[:n_rows].set(idx)
    b_per_w = b_pad // _NW
    mesh = plsc.VectorSubcoreMesh(core_axis_name="c", subcore_axis_name="s")

    @functools.partial(
        pl.kernel, mesh=mesh,
        out_type=jax.ShapeDtypeStruct((b_pad, d), jnp.float32),
        scratch_types=[
            pltpu.VMEM((b_per_w,), jnp.int32),
            pltpu.VMEM((b_per_w, d), jnp.float32),
            pltpu.SemaphoreType.DMA,
        ],
    )
    def k(table_hbm, idx_hbm, out_hbm, idx_v, rows_v, sem):
        wid = lax.axis_index("s") * _NC + lax.axis_index("c")
        base = wid * b_per_w
        pltpu.sync_copy(idx_hbm.at[pl.ds(base, b_per_w)], idx_v)
        pltpu.async_copy(table_hbm.at[idx_v], rows_v, sem).wait()
        pltpu.sync_copy(rows_v, out_hbm.at[pl.ds(base, b_per_w)])

    return k(table, idx_p)[:n_rows]


_GW = 128


def _make_edge_sc(n_src, ls_pad, ld_pad, agg_pad, out, nsplit):
    rows = agg_pad // nsplit
    stripe = rows // _NS
    n_work = _NW // nsplit
    epw = N_EDGES // n_work
    nchunk = epw // _C
    mesh = plsc.VectorSubcoreMesh(core_axis_name="c", subcore_axis_name="s")
    ncol = out // 16

    @functools.partial(
        pl.kernel, mesh=mesh,
        compiler_params=pltpu.CompilerParams(needs_layout_passes=False),
        out_type=[
            jax.ShapeDtypeStruct((_NC, rows, _GW), jnp.float32),
            jax.ShapeDtypeStruct((_NC, rows, 16), jnp.float32),
        ],
        scratch_types=[
            pltpu.VMEM((ls_pad,), jnp.float32),
            pltpu.VMEM((ld_pad,), jnp.float32),
            pltpu.VMEM((48,), jnp.float32),
            pltpu.VMEM((3 * _C,), jnp.int32),
            pltpu.VMEM((_C,), jnp.float32),
            pltpu.VMEM((_C,), jnp.int32),
            pltpu.VMEM((_C,), jnp.int32),
            pltpu.VMEM((_C,), jnp.int32),
            pltpu.VMEM((_C, _GW), jnp.float32),
            pltpu.VMEM((_C, _GW), jnp.float32),
            pltpu.VMEM((_C, 16), jnp.float32),
            pltpu.VMEM_SHARED((rows, _GW), jnp.float32),
            pltpu.VMEM_SHARED((rows, 16), jnp.float32),
            pltpu.SemaphoreType.DMA,
            pltpu.SemaphoreType.DMA,
        ],
    )
    def k(eidx_hbm, u_hbm, ls_hbm, ld_hbm, le_hbm, z1_hbm, z2_hbm,
          out_hbm, den_hbm,
          lstab, ldtab, letab, echunk, exbuf, sbuf, i2buf, dbuf,
          rows_s, rows_e, den_buf, agg_sh, den_sh, sem1, sem2):
        cid = lax.axis_index("c")
        sid = lax.axis_index("s")
        wid = sid * _NC + cid if nsplit == 1 else sid
        lo = jnp.int32(0) if nsplit == 1 else cid * jnp.int32(rows)
        pltpu.sync_copy(ls_hbm, lstab)
        pltpu.sync_copy(ld_hbm, ldtab)
        pltpu.sync_copy(le_hbm, letab)
        pltpu.sync_copy(z1_hbm.at[pl.ds(sid * stripe, stripe)],
                        agg_sh.at[pl.ds(sid * stripe, stripe)])
        pltpu.sync_copy(z2_hbm.at[pl.ds(sid * stripe, stripe)],
                        den_sh.at[pl.ds(sid * stripe, stripe)])
        plsc.subcore_barrier()

        ebase = wid * (nchunk * 3 * _C)

        def p2(c, _):
            pltpu.sync_copy(
                eidx_hbm.at[pl.ds(ebase + c * (3 * _C), 3 * _C)], echunk)

            def g1(g, _):
                sl = pl.ds(g * 16, 16)
                sv = echunk[pl.ds(g * 16, 16)]
                dv = echunk[pl.ds(_C + g * 16, 16)]
                tv = echunk[pl.ds(2 * _C + g * 16, 16)]
                av = plsc.load_gather(lstab, [sv])
                bv = plsc.load_gather(ldtab, [dv])
                cv = plsc.load_gather(letab, [tv])
                l = av + bv + cv
                l = jnp.where(l > 0, l, l * jnp.float32(0.2))
                e = jnp.exp(l)
                dl = dv - lo
                if nsplit > 1:
                    inr = (dl >= 0) & (dl < rows)
                    e = jnp.where(inr, e, jnp.float32(0.0))
                    dl = jnp.clip(dl, 0, rows - 1)
                exbuf[sl] = e
                sbuf[sl] = sv
                dbuf[sl] = dl
                i2buf[sl] = tv + jnp.int32(n_src)
                return 0
            lax.fori_loop(0, _C // 16, g1, 0)

            d1 = pltpu.async_copy(u_hbm.at[sbuf], rows_s, sem1)
            d2 = pltpu.async_copy(u_hbm.at[i2buf], rows_e, sem2)
            d1.wait()
            d2.wait()

            def wrow(i4, _):
                for u in range(4):
                    i = i4 * 4 + u
                    exb = plsc.load_gather(
                        exbuf, [jnp.full((16,), i, jnp.int32)])
                    den_buf[i, :] = exb
                    for j in range(ncol):
                        sl = pl.ds(j * 16, 16)
                        rows_s[i, sl] = exb * (rows_s[i, sl] + rows_e[i, sl])
                return 0
            lax.fori_loop(0, _C // 4, wrow, 0)
            pltpu.sync_copy(rows_s, agg_sh.at[dbuf], add=True)
            pltpu.sync_copy(den_buf, den_sh.at[dbuf], add=True)
            return 0
        lax.fori_loop(0, nchunk, p2, 0)

        plsc.subcore_barrier()
        pltpu.sync_copy(agg_sh.at[pl.ds(sid * stripe, stripe)],
                        out_hbm.at[cid, pl.ds(sid * stripe, stripe)])
        pltpu.sync_copy(den_sh.at[pl.ds(sid * stripe, stripe)],
                        den_hbm.at[cid, pl.ds(sid * stripe, stripe)])

    return k


def _make_prep_tc(n_src, n_dst, ls_pad, ld_pad, rows_pad, out, out_pad):
    del out_pad
    def body(src_ref, dst_ref, wsrc_ref, wdst_ref, we_ref, asrc_ref,
             adst_ref, ae_ref, wst_ref, u_ref, ls_ref, ld_ref, le_ref):
        zs = jnp.dot(src_ref[...], wsrc_ref[...],
                     preferred_element_type=jnp.float32)
        zet = jnp.dot(wst_ref[...], we_ref[...],
                      preferred_element_type=jnp.float32)
        ls = jnp.dot(zs, asrc_ref[...][:, None],
                     preferred_element_type=jnp.float32)
        wv = jnp.dot(wdst_ref[...], adst_ref[...][:, None],
                     preferred_element_type=jnp.float32)
        ld = jnp.dot(dst_ref[...], wv, preferred_element_type=jnp.float32)
        le = jnp.dot(zet, ae_ref[...][:, None],
                     preferred_element_type=jnp.float32)
        if out < _GW:
            zs = jnp.concatenate(
                [zs, jnp.zeros((n_src, _GW - out), jnp.float32)], axis=1)
            zet = jnp.concatenate(
                [zet, jnp.zeros((N_ETYPES, _GW - out), jnp.float32)], axis=1)
        parts = [zs, zet]
        extra = rows_pad - n_src - N_ETYPES
        if extra:
            parts.append(jnp.zeros((extra, _GW), jnp.float32))
        u_ref[...] = jnp.concatenate(parts, axis=0)

        def padrows(x, total):
            extra = total - x.shape[0]
            if not extra:
                return x
            return jnp.concatenate(
                [x, jnp.zeros((extra, 1), jnp.float32)], axis=0)

        ls_ref[...] = padrows(ls, ls_pad)
        ld_ref[...] = padrows(ld, ld_pad)
        le_ref[...] = padrows(le, 48)

    return pl.pallas_call(
        body,
        out_shape=[
            jax.ShapeDtypeStruct((rows_pad, _GW), jnp.float32),
            jax.ShapeDtypeStruct((ls_pad, 1), jnp.float32),
            jax.ShapeDtypeStruct((ld_pad, 1), jnp.float32),
            jax.ShapeDtypeStruct((48, 1), jnp.float32),
        ],
    )


def _make_finish_tc(n_dst, agg_pad, out, nsplit):
    def body(agg_ref, den_ref, wf1_ref, wf2_ref, out_ref):
        if nsplit == 1:
            a = agg_ref[0] + agg_ref[1]
            den = (den_ref[0] + den_ref[1])[:, 0:1]
        else:
            a = jnp.concatenate([agg_ref[0], agg_ref[1]], axis=0)
            den = jnp.concatenate([den_ref[0], den_ref[1]], axis=0)[:, 0:1]
        g = a[:, :out] / (den + jnp.float32(1e-9))
        h = jnp.where(g > 0, g, jnp.exp(g) - jnp.float32(1.0))
        f = jnp.maximum(
            jnp.dot(h, wf1_ref[...], preferred_element_type=jnp.float32),
            jnp.float32(0.0))
        out_ref[...] = h + jnp.dot(f, wf2_ref[...],
                                   preferred_element_type=jnp.float32)

    return pl.pallas_call(
        body,
        out_shape=jax.ShapeDtypeStruct((agg_pad, out), jnp.float32))


def _sent_proj(sent_feature, W_sent):
    def body(x_ref, w_ref, o_ref):
        o_ref[...] = jnp.dot(x_ref[...], w_ref[...],
                             preferred_element_type=jnp.float32)
    return pl.pallas_call(
        body, out_shape=jax.ShapeDtypeStruct((N_SENTS, HID), jnp.float32)
    )(sent_feature, W_sent)


_S2W = dict(n_src=N_SENTS, n_dst=N_WORDS, ls_pad=512, ld_pad=N_WORDS,
            agg_pad=10240, rows_pad=544, out=EMBED, nsplit=2)
_W2S = dict(n_src=N_WORDS, n_dst=N_SENTS, ls_pad=N_WORDS, ld_pad=512,
            agg_pad=512, rows_pad=N_WORDS + 48, out=HID, nsplit=1)

_EDGE_SC = {k: _make_edge_sc(c["n_src"], c["ls_pad"], c["ld_pad"],
                             c["agg_pad"], c["out"], c["nsplit"])
            for k, c in (("s2w", _S2W), ("w2s", _W2S))}
_PREP_TC = {k: _make_prep_tc(c["n_src"], c["n_dst"], c["ls_pad"], c["ld_pad"],
                             c["rows_pad"], c["out"], None)
            for k, c in (("s2w", _S2W), ("w2s", _W2S))}
_FIN_TC = {k: _make_finish_tc(c["n_dst"], c["agg_pad"], c["out"], c["nsplit"])
           for k, c in (("s2w", _S2W), ("w2s", _W2S))}


def _edge_slab(src_idx, dst_idx, lnk, nsplit):
    n_work = _NW // nsplit
    nchunk = N_EDGES // n_work // _C
    e = jnp.stack([src_idx, dst_idx, lnk], axis=0)
    e = e.reshape(3, n_work, nchunk, _C).transpose(1, 2, 0, 3)
    return e.reshape(-1)


def _gat_pass(kind, cfg, src_state, dst_state, p, eidx, z1, z2, ws_table):
    u, ls, ld, le = _PREP_TC[kind](
        src_state, dst_state, p["Wsrc"], p["Wdst"], p["We"],
        p["a_src"], p["a_dst"], p["a_e"], ws_table)
    agg2, den2 = _EDGE_SC[kind](eidx, u, ls.reshape(-1), ld.reshape(-1),
                                le.reshape(-1), z1, z2)
    out = _FIN_TC[kind](agg2, den2, p["Wf1"], p["Wf2"])
    return out[:cfg["n_dst"]]


def kernel(word_ids, edge_src, edge_dst, ws_link, sent_feature, word_table,
           ws_table, W_sent, p_w2s, p_s2w):
    word_state = _sc_gather_rows(word_table, word_ids, N_WORDS, EMBED)
    sent_state = _sent_proj(sent_feature, W_sent)
    eidx_w2s = _edge_slab(edge_src, edge_dst, ws_link, _W2S["nsplit"])
    eidx_s2w = _edge_slab(edge_dst, edge_src, ws_link, _S2W["nsplit"])
    z1_s2w = jnp.zeros((_S2W["agg_pad"] // _S2W["nsplit"], _GW), jnp.float32)
    z2_s2w = jnp.zeros((_S2W["agg_pad"] // _S2W["nsplit"], 16), jnp.float32)
    z1_w2s = jnp.zeros((_W2S["agg_pad"], _GW), jnp.float32)
    z2_w2s = jnp.zeros((_W2S["agg_pad"], 16), jnp.float32)

    word_state = _gat_pass("s2w", _S2W, sent_state, word_state, p_s2w,
                           eidx_s2w, z1_s2w, z2_s2w, ws_table)

    def it(_, carry):
        ws, ss = carry
        ss = _gat_pass("w2s", _W2S, ws, ss, p_w2s,
                       eidx_w2s, z1_w2s, z2_w2s, ws_table)
        ws = _gat_pass("s2w", _S2W, ss, ws, p_s2w,
                       eidx_s2w, z1_s2w, z2_s2w, ws_table)
        return (ws, ss)

    word_state, sent_state = lax.fori_loop(
        0, N_ITER, it, (word_state, sent_state))
    return word_state
